# baseline (speedup 1.0000x reference)
import jax
import jax.numpy as jnp
from jax.experimental import pallas as pl
from jax.experimental.pallas import tpu as pltpu

BM = 128
BN = 64
NKM = BM // 4
NKN = BN // 4
LN2 = 0.6931471805599453
NEG = -1e30


def _mmx(a, b):
    return jax.lax.dot_general(a, b, (((1,), (0,)), ((), ())),
                               preferred_element_type=jnp.float32,
                               precision=jax.lax.Precision.HIGHEST)


def _mmd(a, b):
    return jax.lax.dot_general(a, b, (((1,), (0,)), ((), ())),
                               preferred_element_type=jnp.float32)


def _mmTd(a, b):
    return jax.lax.dot_general(a, b, (((1,), (1,)), ((), ())),
                               preferred_element_type=jnp.float32)


def _mask_kernel(q_ref, k_ref, selq_ref, selk_ref, ids_ref):
    L = q_ref.shape[2]
    nbq, nbk = L // BM, L // BN
    selq = selq_ref[0, 0]
    selk = selk_ref[0, 0]
    sq = jnp.concatenate(
        [_mmd(selq, q_ref[0, 0, i * BM:(i + 1) * BM, :]) for i in range(nbq)], 0)
    sk = jnp.concatenate(
        [_mmd(selk, k_ref[0, 0, j * BN:(j + 1) * BN, :]) for j in range(nbk)], 0)
    scale = 1.0 / (q_ref.shape[3] ** 0.5)
    logits = _mmTd(sq, sk) * scale
    m = jnp.max(logits, axis=1, keepdims=True)
    e = jnp.exp(logits - m)
    probs = e / jnp.sum(e, axis=1, keepdims=True)
    r = jax.lax.broadcasted_iota(jnp.int32, (nbq * NKM, nbk), 0)
    c = jax.lax.broadcasted_iota(jnp.int32, (nbq * NKM, nbk), 1)
    sumk = jnp.where(r // NKN == c, 1.0, 0.0)
    p_hi = probs.astype(jnp.bfloat16).astype(jnp.float32)
    r1 = probs - p_hi
    p_mid = r1.astype(jnp.bfloat16).astype(jnp.float32)
    p_lo = r1 - p_mid
    pk = _mmd(p_hi, sumk) + _mmd(p_mid, sumk) + _mmd(p_lo, sumk)
    r2 = jax.lax.broadcasted_iota(jnp.int32, (nbq * NKM, nbq), 0)
    c2 = jax.lax.broadcasted_iota(jnp.int32, (nbq * NKM, nbq), 1)
    meanq = jnp.where(r2 // NKM == c2, 1.0 / NKM, 0.0)
    pooling = jax.lax.dot_general(meanq, pk, (((0,), (0,)), ((), ())),
                                  preferred_element_type=jnp.float32,
                                  precision=jax.lax.Precision.HIGHEST)
    col = jax.lax.broadcasted_iota(jnp.int32, (nbq, nbk), 1)
    rank = jnp.zeros((nbq, nbk), jnp.int32)
    for jp in range(nbk):
        cv = pooling[:, jp:jp + 1]
        rank += (cv > pooling).astype(jnp.int32)
        rank += ((cv == pooling) & (jp < col)).astype(jnp.int32)
    ids = jnp.zeros((nbq, nbk), jnp.int32)
    for j in range(nbk):
        ids += j * (rank[:, j:j + 1] == col).astype(jnp.int32)
    ids_ref[0, 0] = ids


QPG = 16


def _attn_kernel(ids_ref, q_ref, k_ref, v_ref, o_ref):
    b = pl.program_id(0)
    h = pl.program_id(1)
    qg = pl.program_id(2)
    D = q_ref.shape[3]
    scale = 1.0 / (D ** 0.5)
    r8 = jax.lax.broadcasted_iota(jnp.int32, (8, BN), 0)
    c8 = jax.lax.broadcasted_iota(jnp.int32, (8, BN), 1)
    P8 = jnp.where(c8 // 8 == r8, 0.125, 0.0)
    r2 = jax.lax.broadcasted_iota(jnp.int32, (BN // 2, BN), 0)
    c2 = jax.lax.broadcasted_iota(jnp.int32, (BN // 2, BN), 1)
    P2 = jnp.where(c2 // 2 == r2, 0.5, 0.0)
    ci = jax.lax.broadcasted_iota(jnp.int32, (1, 384), 1)
    bias = jnp.where(ci < 256, 0.0,
                     jnp.where(ci < 288, LN2,
                               jnp.where(ci < 344, 3.0 * LN2, NEG)))
    zpad = jnp.zeros((40, BN), jnp.float32)

    for qq in range(QPG):
        qb = qg * QPG + qq
        q = q_ref[0, 0, qq * BM:(qq + 1) * BM, :]
        kbs = []
        vbs = []
        for s in range(12):
            bid = ids_ref[b, h, qb, s]
            kbs.append(k_ref[0, 0, pl.ds(bid * BN, BN), :])
            vbs.append(v_ref[0, 0, pl.ds(bid * BN, BN), :])
        ksel = jnp.concatenate(
            kbs[:4] + [_mmd(P2, kbs[4])] + [_mmd(P8, kbs[s]) for s in range(5, 12)]
            + [zpad], 0)
        vsel = jnp.concatenate(
            vbs[:4] + [_mmd(P2, vbs[4])] + [_mmd(P8, vbs[s]) for s in range(5, 12)]
            + [zpad], 0)
        s = _mmTd(q, ksel) * scale + bias
        m = jnp.max(s, axis=1, keepdims=True)
        e = jnp.exp(s - m)
        l = jnp.sum(e, axis=1, keepdims=True)
        o_ref[0, 0, qq * BM:(qq + 1) * BM, :] = _mmd(e, vsel) / l


def _sample_onehots(B, H):
    key = jax.random.key(42)
    k1, k2 = jax.random.split(key)
    rvq = jax.random.uniform(k1, (B, H, 1, BM))
    _, idxq = jax.lax.top_k(rvq, NKM)
    rvk = jax.random.uniform(k2, (B, H, 1, BN))
    _, idxk = jax.lax.top_k(rvk, NKN)
    selq = jax.nn.one_hot(idxq[:, :, 0, :], BM, dtype=jnp.float32)
    selk = jax.nn.one_hot(idxk[:, :, 0, :], BN, dtype=jnp.float32)
    return selq, selk


def kernel(q, k, v):
    B, H, L, D = q.shape
    selq, selk = _sample_onehots(B, H)
    nbq, nbk = L // BM, L // BN

    ids = pl.pallas_call(
        _mask_kernel,
        grid=(B, H),
        in_specs=[
            pl.BlockSpec((1, 1, L, D), lambda b, h: (b, h, 0, 0)),
            pl.BlockSpec((1, 1, L, D), lambda b, h: (b, h, 0, 0)),
            pl.BlockSpec((1, 1, NKM, BM), lambda b, h: (b, h, 0, 0)),
            pl.BlockSpec((1, 1, NKN, BN), lambda b, h: (b, h, 0, 0)),
        ],
        out_specs=pl.BlockSpec((1, 1, nbq, nbk), lambda b, h: (b, h, 0, 0)),
        out_shape=jax.ShapeDtypeStruct((B, H, nbq, nbk), jnp.int32),
    )(q, k, selq, selk)

    grid_spec = pltpu.PrefetchScalarGridSpec(
        num_scalar_prefetch=1,
        grid=(B, H, nbq // QPG),
        in_specs=[
            pl.BlockSpec((1, 1, QPG * BM, D), lambda b, h, qg, ids_r: (b, h, qg, 0)),
            pl.BlockSpec((1, 1, L, D), lambda b, h, qg, ids_r: (b, h, 0, 0)),
            pl.BlockSpec((1, 1, L, D), lambda b, h, qg, ids_r: (b, h, 0, 0)),
        ],
        out_specs=pl.BlockSpec((1, 1, QPG * BM, D), lambda b, h, qg, ids_r: (b, h, qg, 0)),
    )
    out = pl.pallas_call(
        _attn_kernel,
        grid_spec=grid_spec,
        out_shape=jax.ShapeDtypeStruct((B, H, L, D), jnp.float32),
    )(ids, q, k, v)
    return out
